# baseline (speedup 1.0000x reference)
import functools

import jax
import jax.numpy as jnp
from jax import lax
from jax._src import config as _config
from jax.experimental import pallas as pl
from jax.experimental.pallas import tpu as pltpu
from jax.experimental.pallas import tpu_sc as plsc

N = 10000
E = 320000
D = 128
H = 8
DH = 16
ALPHA = 0.2

NC = 2
NS = 16
NW = NC * NS
EPW = E // NW
C = 80
NCHUNK = EPW // C
NPAD = 10240
ROWS_PER_TILE = NPAD // NS
TD = 144


def _proj_body(x_ref, w_ref, a1_ref, a2_ref, td_ref, ts_ref):
    xb = x_ref[...]
    hf = jnp.dot(xb, w_ref[...], preferred_element_type=jnp.float32)
    s1 = jnp.dot(hf, a1_ref[...], preferred_element_type=jnp.float32)
    s2 = jnp.dot(hf, a2_ref[...], preferred_element_type=jnp.float32)
    z8 = jnp.zeros((xb.shape[0], 8), jnp.float32)
    td_ref[...] = jnp.concatenate([hf, s2, z8], axis=1)
    ts_ref[...] = jnp.concatenate([s1, z8], axis=1)


def _finish_body(p_ref, r_ref, node_ref):
    s = p_ref[0] + p_ref[1]
    hp = s[:, :128]
    rs = s[:, 128:136]
    denom = jnp.dot(rs, r_ref[...], preferred_element_type=jnp.float32) + 1e-16
    v = hp / denom
    node_ref[...] = jnp.where(v > 0, v, jnp.exp(v) - 1.0)


BN = 1000


def _tc_proj(x, wflat, a1m, a2m):
    return pl.pallas_call(
        _proj_body,
        grid=(N // BN,),
        in_specs=[
            pl.BlockSpec((BN, D), lambda i: (i, 0)),
            pl.BlockSpec((D, D), lambda i: (0, 0)),
            pl.BlockSpec((D, H), lambda i: (0, 0)),
            pl.BlockSpec((D, H), lambda i: (0, 0)),
        ],
        out_specs=[
            pl.BlockSpec((BN, TD), lambda i: (i, 0)),
            pl.BlockSpec((BN, 16), lambda i: (i, 0)),
        ],
        out_shape=[
            jax.ShapeDtypeStruct((N, TD), jnp.float32),
            jax.ShapeDtypeStruct((N, 16), jnp.float32),
        ],
    )(x, wflat, a1m, a2m)


def _tc_finish(p, rmat):
    return pl.pallas_call(
        _finish_body,
        grid=(N // BN,),
        in_specs=[
            pl.BlockSpec((2, BN, TD), lambda i: (0, i, 0)),
            pl.BlockSpec((H, D), lambda i: (0, 0)),
        ],
        out_specs=pl.BlockSpec((BN, D), lambda i: (i, 0)),
        out_shape=jax.ShapeDtypeStruct((N, D), jnp.float32),
    )(p, rmat)


def _sc_edges_body(src_hbm, dst_hbm, td_hbm, ts_hbm, zero_hbm, out_hbm,
                   s_idx0, d_idx0, rdst0, rsrc0,
                   s_idx1, d_idx1, rdst1, rsrc1,
                   wrow, acc, semd0, sems0, semd1, sems1):
    cid = lax.axis_index("c")
    sid = lax.axis_index("s")
    wid = cid * jnp.int32(NS) + sid
    row0 = sid * jnp.int32(ROWS_PER_TILE)

    bufs = ((s_idx0, d_idx0, rdst0, rsrc0, semd0, sems0),
            (s_idx1, d_idx1, rdst1, rsrc1, semd1, sems1))

    def issue(k, b):
        s_idx, d_idx, rdst, rsrc, semd, sems = bufs[b]
        base = wid * jnp.int32(EPW) + k * jnp.int32(C)
        pltpu.sync_copy(src_hbm.at[pl.ds(base, C)], s_idx)
        pltpu.sync_copy(dst_hbm.at[pl.ds(base, C)], d_idx)
        pltpu.async_copy(td_hbm.at[d_idx], rdst, semd)
        pltpu.async_copy(ts_hbm.at[s_idx], rsrc, sems)

    def consume(b):
        s_idx, d_idx, rdst, rsrc, semd, sems = bufs[b]
        pltpu.make_async_copy(td_hbm.at[d_idx], rdst, semd).wait()
        pltpu.make_async_copy(ts_hbm.at[s_idx], rsrc, sems).wait()

        @plsc.parallel_loop(0, C, unroll=2)
        def edge(i):
            v2 = rdst[i, pl.ds(128, 16)]
            v1 = rsrc[i, :]
            t = v1 + v2
            lr = jnp.where(t >= 0, t, ALPHA * t)
            e = jnp.exp(-lr)
            wrow[i, pl.ds(128, 16)] = e
            for h in range(H):
                eh = e[h]
                wrow[i, pl.ds(h * 16, 16)] = rdst[i, pl.ds(h * 16, 16)] * eh
        pltpu.sync_copy(wrow, acc.at[s_idx], add=True)

    pltpu.sync_copy(zero_hbm, acc.at[pl.ds(row0, ROWS_PER_TILE)])
    plsc.subcore_barrier()

    issue(jnp.int32(0), 0)

    def body2(j, carry):
        k = j * jnp.int32(2)

        @pl.when(k + 1 < NCHUNK)
        def _():
            issue(k + 1, 1)

        consume(0)

        @pl.when(k + 2 < NCHUNK)
        def _():
            issue(k + 2, 0)

        @pl.when(k + 1 < NCHUNK)
        def _():
            consume(1)

        return carry

    lax.fori_loop(jnp.int32(0), jnp.int32((NCHUNK + 1) // 2), body2,
                  jnp.int32(0))
    plsc.subcore_barrier()
    pltpu.sync_copy(acc.at[pl.ds(row0, ROWS_PER_TILE)],
                    out_hbm.at[cid, pl.ds(row0, ROWS_PER_TILE)])


_sc_edges = functools.partial(
    pl.kernel,
    out_type=jax.ShapeDtypeStruct((NC, NPAD, TD), jnp.float32),
    mesh=plsc.VectorSubcoreMesh(core_axis_name="c", subcore_axis_name="s"),
    compiler_params=pltpu.CompilerParams(use_tc_tiling_on_sc=False),
    scratch_types=[
        pltpu.VMEM((C,), jnp.int32),
        pltpu.VMEM((C,), jnp.int32),
        pltpu.VMEM((C, TD), jnp.float32),
        pltpu.VMEM((C, 16), jnp.float32),
        pltpu.VMEM((C,), jnp.int32),
        pltpu.VMEM((C,), jnp.int32),
        pltpu.VMEM((C, TD), jnp.float32),
        pltpu.VMEM((C, 16), jnp.float32),
        pltpu.VMEM((C, TD), jnp.float32),
        pltpu.VMEM_SHARED((NPAD, TD), jnp.float32),
        pltpu.SemaphoreType.DMA,
        pltpu.SemaphoreType.DMA,
        pltpu.SemaphoreType.DMA,
        pltpu.SemaphoreType.DMA,
    ],
)(_sc_edges_body)


def _layer_tables(node, Wl, al):
    wflat = jnp.transpose(Wl, (1, 0, 2)).reshape(D, H * DH)
    a1 = al[:, :DH]
    a2 = al[:, DH:]
    eye = jnp.eye(H, dtype=jnp.float32)
    a1m = (a1[:, :, None] * eye[:, None, :]).reshape(H * DH, H)
    a2m = (a2[:, :, None] * eye[:, None, :]).reshape(H * DH, H)
    return _tc_proj(node, wflat, a1m, a2m)


def kernel(x, adj, W, a):
    with _config.enable_x64(False):
        x = x.astype(jnp.float32)
        src = adj[0].astype(jnp.int32)
        dst = adj[1].astype(jnp.int32)
        W = W.astype(jnp.float32)
        a = a.astype(jnp.float32)
        zero = jnp.zeros((ROWS_PER_TILE, TD), jnp.float32)
        rmat = jnp.repeat(jnp.eye(H, dtype=jnp.float32), DH, axis=1)

        node = x
        for l in range(2):
            td, ts = _layer_tables(node, W[l], a[l])
            p = _sc_edges(src, dst, td, ts, zero)
            node = _tc_finish(p, rmat)
        return node


# speedup vs baseline: 1.2692x; 1.2692x over previous
import functools

import jax
import jax.numpy as jnp
import numpy as _np
from jax import lax
from jax._src import config as _config
from jax.experimental import pallas as pl
from jax.experimental.pallas import tpu as pltpu
from jax.experimental.pallas import tpu_sc as plsc

N = 10000
E = 320000
D = 128
H = 8
DH = 16
ALPHA = 0.2

NC = 2
NS = 16
NW = NC * NS
EPW = E // NW
C = 40
NCHUNK = EPW // C
NPAD = 10240
ROWS_PER_TILE = NPAD // NS
TD = 144

_PERM = _np.empty(128, dtype=_np.int32)
for _p in range(4):
    for _j in range(16):
        _PERM[32 * _p + 2 * _j] = 32 * _p + _j
        _PERM[32 * _p + 2 * _j + 1] = 32 * _p + 16 + _j


def _proj_body(x_ref, w_ref, a1_ref, a2_ref, tdf_ref, ts1_ref, ts2_ref):
    xb = x_ref[...]
    hf = jnp.dot(xb, w_ref[...], preferred_element_type=jnp.float32)
    s1 = jnp.dot(hf, a1_ref[...], preferred_element_type=jnp.float32)
    s2 = jnp.dot(hf, a2_ref[...], preferred_element_type=jnp.float32)
    z8 = jnp.zeros((xb.shape[0], 8), jnp.float32)
    tdf_ref[...] = hf.astype(jnp.bfloat16)
    ts1_ref[...] = jnp.concatenate([s1, z8], axis=1)
    ts2_ref[...] = jnp.concatenate([s2, z8], axis=1)


def _finish_body(p_ref, r_ref, node_ref):
    s = p_ref[0] + p_ref[1]
    hp = s[:, :128]
    rs = s[:, 128:136]
    denom = jnp.dot(rs, r_ref[...], preferred_element_type=jnp.float32) + 1e-16
    v = hp / denom
    node_ref[...] = jnp.where(v > 0, v, jnp.exp(v) - 1.0)


BN = 1000


def _tc_proj(x, wflat, a1m, a2m):
    return pl.pallas_call(
        _proj_body,
        grid=(N // BN,),
        in_specs=[
            pl.BlockSpec((BN, D), lambda i: (i, 0)),
            pl.BlockSpec((D, D), lambda i: (0, 0)),
            pl.BlockSpec((D, H), lambda i: (0, 0)),
            pl.BlockSpec((D, H), lambda i: (0, 0)),
        ],
        out_specs=[
            pl.BlockSpec((BN, D), lambda i: (i, 0)),
            pl.BlockSpec((BN, 16), lambda i: (i, 0)),
            pl.BlockSpec((BN, 16), lambda i: (i, 0)),
        ],
        out_shape=[
            jax.ShapeDtypeStruct((N, D), jnp.bfloat16),
            jax.ShapeDtypeStruct((N, 16), jnp.float32),
            jax.ShapeDtypeStruct((N, 16), jnp.float32),
        ],
    )(x, wflat, a1m, a2m)


def _tc_finish(p, rmat):
    return pl.pallas_call(
        _finish_body,
        grid=(N // BN,),
        in_specs=[
            pl.BlockSpec((2, BN, TD), lambda i: (0, i, 0)),
            pl.BlockSpec((H, D), lambda i: (0, 0)),
        ],
        out_specs=pl.BlockSpec((BN, D), lambda i: (i, 0)),
        out_shape=jax.ShapeDtypeStruct((N, D), jnp.float32),
    )(p, rmat)


def _sc_edges_body(src_hbm, dst_hbm, tdf_hbm, ts1_hbm, ts2_hbm, zero_hbm,
                   out_hbm,
                   s_idx0, d_idx0, rdf0, rs1_0, rs2_0, wrow0, c_idx0,
                   s_idx1, d_idx1, rdf1, rs1_1, rs2_1, wrow1, c_idx1,
                   acc,
                   semi0, semf0, sem10, sem20, semw0,
                   semi1, semf1, sem11, sem21, semw1):
    cid = lax.axis_index("c")
    sid = lax.axis_index("s")
    wid = cid * jnp.int32(NS) + sid
    row0 = sid * jnp.int32(ROWS_PER_TILE)

    bufs = ((s_idx0, d_idx0, rdf0, rs1_0, rs2_0, wrow0, c_idx0,
             semi0, semf0, sem10, sem20, semw0),
            (s_idx1, d_idx1, rdf1, rs1_1, rs2_1, wrow1, c_idx1,
             semi1, semf1, sem11, sem21, semw1))

    def issue_idx(k, b):
        s_idx, d_idx = bufs[b][0], bufs[b][1]
        semi = bufs[b][7]
        base = wid * jnp.int32(EPW) + k * jnp.int32(C)
        pltpu.async_copy(src_hbm.at[pl.ds(base, C)], s_idx, semi)
        pltpu.async_copy(dst_hbm.at[pl.ds(base, C)], d_idx, semi)

    def wait_idx(b):
        s_idx, d_idx = bufs[b][0], bufs[b][1]
        semi = bufs[b][7]
        pltpu.make_async_copy(src_hbm.at[pl.ds(0, C)], s_idx, semi).wait()
        pltpu.make_async_copy(dst_hbm.at[pl.ds(0, C)], d_idx, semi).wait()

    def issue_rows(b):
        s_idx, d_idx, rdf, rs1, rs2 = bufs[b][:5]
        semf, sem1, sem2 = bufs[b][8], bufs[b][9], bufs[b][10]
        pltpu.async_copy(tdf_hbm.at[d_idx], rdf, semf)
        pltpu.async_copy(ts1_hbm.at[s_idx], rs1, sem1)
        pltpu.async_copy(ts2_hbm.at[d_idx], rs2, sem2)

    def wait_rows(b):
        s_idx, d_idx, rdf, rs1, rs2 = bufs[b][:5]
        semf, sem1, sem2 = bufs[b][8], bufs[b][9], bufs[b][10]
        pltpu.make_async_copy(tdf_hbm.at[d_idx], rdf, semf).wait()
        pltpu.make_async_copy(ts1_hbm.at[s_idx], rs1, sem1).wait()
        pltpu.make_async_copy(ts2_hbm.at[d_idx], rs2, sem2).wait()

    def wait_scatter(b):
        wrow, c_idx, semw = bufs[b][5], bufs[b][6], bufs[b][11]
        pltpu.make_async_copy(wrow, acc.at[c_idx], semw).wait()

    def copy_scatter_idx(b):
        s_idx, c_idx = bufs[b][0], bufs[b][6]
        c_idx[pl.ds(0, 16)] = s_idx[pl.ds(0, 16)]
        c_idx[pl.ds(16, 16)] = s_idx[pl.ds(16, 16)]
        c_idx[pl.ds(24, 16)] = s_idx[pl.ds(24, 16)]

    def compute_scatter(b):
        _, _, rdf, rs1, rs2, wrow, c_idx = bufs[b][:7]
        semw = bufs[b][11]

        @plsc.parallel_loop(0, C)
        def edge(i):
            t = rs1[i, :] + rs2[i, :]
            lr = jnp.where(t >= 0, t, ALPHA * t)
            e = jnp.exp(-lr)
            wrow[i, pl.ds(128, 16)] = e
            for p in range(4):
                pair = rdf[i, pl.ds(p * 32, 32)]
                fa, fb = plsc.unpack(pair, format=plsc.PackFormat.INTERLEAVED)
                wrow[i, pl.ds(p * 32, 16)] = fa * e[2 * p]
                wrow[i, pl.ds(p * 32 + 16, 16)] = fb * e[2 * p + 1]

        pltpu.async_copy(wrow, acc.at[c_idx], semw, add=True)

    pltpu.sync_copy(zero_hbm, acc.at[pl.ds(row0, ROWS_PER_TILE)])
    plsc.subcore_barrier()

    issue_idx(jnp.int32(0), 0)
    issue_idx(jnp.int32(1), 1)
    wait_idx(0)
    issue_rows(0)

    def body2(j, carry):
        k = j * jnp.int32(2)

        wait_idx(1)
        issue_rows(1)
        wait_rows(0)

        @pl.when(j > 0)
        def _():
            wait_scatter(0)

        copy_scatter_idx(0)

        @pl.when(k + 2 < NCHUNK)
        def _():
            issue_idx(k + 2, 0)

        compute_scatter(0)

        @pl.when(k + 2 < NCHUNK)
        def _():
            wait_idx(0)
            issue_rows(0)

        wait_rows(1)

        @pl.when(j > 0)
        def _():
            wait_scatter(1)

        copy_scatter_idx(1)

        @pl.when(k + 3 < NCHUNK)
        def _():
            issue_idx(k + 3, 1)

        compute_scatter(1)
        return carry

    lax.fori_loop(jnp.int32(0), jnp.int32(NCHUNK // 2), body2, jnp.int32(0))
    wait_scatter(0)
    wait_scatter(1)
    plsc.subcore_barrier()
    pltpu.sync_copy(acc.at[pl.ds(row0, ROWS_PER_TILE)],
                    out_hbm.at[cid, pl.ds(row0, ROWS_PER_TILE)])


_sc_edges = functools.partial(
    pl.kernel,
    out_type=jax.ShapeDtypeStruct((NC, NPAD, TD), jnp.float32),
    mesh=plsc.VectorSubcoreMesh(core_axis_name="c", subcore_axis_name="s"),
    compiler_params=pltpu.CompilerParams(use_tc_tiling_on_sc=False,
                                         needs_layout_passes=False),
    scratch_types=[
        pltpu.VMEM((C,), jnp.int32),
        pltpu.VMEM((C,), jnp.int32),
        pltpu.VMEM((C, D), jnp.bfloat16),
        pltpu.VMEM((C, 16), jnp.float32),
        pltpu.VMEM((C, 16), jnp.float32),
        pltpu.VMEM((C, TD), jnp.float32),
        pltpu.VMEM((C,), jnp.int32),
        pltpu.VMEM((C,), jnp.int32),
        pltpu.VMEM((C,), jnp.int32),
        pltpu.VMEM((C, D), jnp.bfloat16),
        pltpu.VMEM((C, 16), jnp.float32),
        pltpu.VMEM((C, 16), jnp.float32),
        pltpu.VMEM((C, TD), jnp.float32),
        pltpu.VMEM((C,), jnp.int32),
        pltpu.VMEM_SHARED((NPAD, TD), jnp.float32),
        pltpu.SemaphoreType.DMA,
        pltpu.SemaphoreType.DMA,
        pltpu.SemaphoreType.DMA,
        pltpu.SemaphoreType.DMA,
        pltpu.SemaphoreType.DMA,
        pltpu.SemaphoreType.DMA,
        pltpu.SemaphoreType.DMA,
        pltpu.SemaphoreType.DMA,
        pltpu.SemaphoreType.DMA,
        pltpu.SemaphoreType.DMA,
    ],
)(_sc_edges_body)


def _layer_tables(node, Wl, al):
    wflat = jnp.transpose(Wl, (1, 0, 2)).reshape(D, H * DH)[:, _PERM]
    a1 = al[:, :DH]
    a2 = al[:, DH:]
    eye = jnp.eye(H, dtype=jnp.float32)
    a1m = (a1[:, :, None] * eye[:, None, :]).reshape(H * DH, H)[_PERM, :]
    a2m = (a2[:, :, None] * eye[:, None, :]).reshape(H * DH, H)[_PERM, :]
    return _tc_proj(node, wflat, a1m, a2m)


def kernel(x, adj, W, a):
    with _config.enable_x64(False):
        x = x.astype(jnp.float32)
        src = adj[0].astype(jnp.int32)
        dst = adj[1].astype(jnp.int32)
        W = W.astype(jnp.float32)
        a = a.astype(jnp.float32)
        zero = jnp.zeros((ROWS_PER_TILE, TD), jnp.float32)
        rmat = jnp.repeat(jnp.eye(H, dtype=jnp.float32), DH, axis=1)

        node = x
        for l in range(2):
            tdf, ts1, ts2 = _layer_tables(node, W[l], a[l])
            p = _sc_edges(src, dst, tdf, ts1, ts2, zero)
            node = _tc_finish(p, rmat)
        return node


# speedup vs baseline: 1.5729x; 1.2393x over previous
import functools

import jax
import jax.numpy as jnp
import numpy as _np
from jax import lax
from jax._src import config as _config
from jax.experimental import pallas as pl
from jax.experimental.pallas import tpu as pltpu
from jax.experimental.pallas import tpu_sc as plsc

N = 10000
E = 320000
D = 128
H = 8
DH = 16
ALPHA = 0.2

NC = 2
NS = 16
NW = NC * NS
EPW = E // NW
C = 80
NCHUNK = EPW // C
NPAD = 10240
ROWS_PER_TILE = NPAD // NS
TD = 144

_PERM = _np.empty(128, dtype=_np.int32)
for _p in range(4):
    for _j in range(16):
        _PERM[32 * _p + 2 * _j] = 32 * _p + _j
        _PERM[32 * _p + 2 * _j + 1] = 32 * _p + 16 + _j


def _proj_body(x_ref, w_ref, a1_ref, a2_ref, tdf_ref, ts1_ref, ts2_ref):
    xb = x_ref[...]
    hf = jnp.dot(xb, w_ref[...], preferred_element_type=jnp.float32)
    s1 = jnp.dot(hf, a1_ref[...], preferred_element_type=jnp.float32)
    s2 = jnp.dot(hf, a2_ref[...], preferred_element_type=jnp.float32)
    z8 = jnp.zeros((xb.shape[0], 8), jnp.float32)
    tdf_ref[...] = hf.astype(jnp.bfloat16)
    ts1_ref[...] = jnp.concatenate([s1, z8], axis=1)
    ts2_ref[...] = jnp.concatenate([s2, z8], axis=1)


def _finish_body(p_ref, r_ref, node_ref):
    s = p_ref[0] + p_ref[1]
    hp = s[:, :128]
    rs = s[:, 128:136]
    denom = jnp.dot(rs, r_ref[...], preferred_element_type=jnp.float32) + 1e-16
    v = hp / denom
    node_ref[...] = jnp.where(v > 0, v, jnp.exp(v) - 1.0)


def _finish_proj_body(p_ref, r_ref, w_ref, a1_ref, a2_ref,
                      tdf_ref, ts1_ref, ts2_ref):
    s = p_ref[0] + p_ref[1]
    hp = s[:, :128]
    rs = s[:, 128:136]
    denom = jnp.dot(rs, r_ref[...], preferred_element_type=jnp.float32) + 1e-16
    v = hp / denom
    node = jnp.where(v > 0, v, jnp.exp(v) - 1.0)
    hf = jnp.dot(node, w_ref[...], preferred_element_type=jnp.float32)
    s1 = jnp.dot(hf, a1_ref[...], preferred_element_type=jnp.float32)
    s2 = jnp.dot(hf, a2_ref[...], preferred_element_type=jnp.float32)
    z8 = jnp.zeros((node.shape[0], 8), jnp.float32)
    tdf_ref[...] = hf.astype(jnp.bfloat16)
    ts1_ref[...] = jnp.concatenate([s1, z8], axis=1)
    ts2_ref[...] = jnp.concatenate([s2, z8], axis=1)


BN = 1000


def _tc_proj(x, wflat, a1m, a2m):
    return pl.pallas_call(
        _proj_body,
        grid=(N // BN,),
        in_specs=[
            pl.BlockSpec((BN, D), lambda i: (i, 0)),
            pl.BlockSpec((D, D), lambda i: (0, 0)),
            pl.BlockSpec((D, H), lambda i: (0, 0)),
            pl.BlockSpec((D, H), lambda i: (0, 0)),
        ],
        out_specs=[
            pl.BlockSpec((BN, D), lambda i: (i, 0)),
            pl.BlockSpec((BN, 16), lambda i: (i, 0)),
            pl.BlockSpec((BN, 16), lambda i: (i, 0)),
        ],
        out_shape=[
            jax.ShapeDtypeStruct((N, D), jnp.bfloat16),
            jax.ShapeDtypeStruct((N, 16), jnp.float32),
            jax.ShapeDtypeStruct((N, 16), jnp.float32),
        ],
    )(x, wflat, a1m, a2m)


def _tc_finish_proj(p, rmat, wflat, a1m, a2m):
    return pl.pallas_call(
        _finish_proj_body,
        grid=(N // BN,),
        in_specs=[
            pl.BlockSpec((2, BN, TD), lambda i: (0, i, 0)),
            pl.BlockSpec((H, D), lambda i: (0, 0)),
            pl.BlockSpec((D, D), lambda i: (0, 0)),
            pl.BlockSpec((D, H), lambda i: (0, 0)),
            pl.BlockSpec((D, H), lambda i: (0, 0)),
        ],
        out_specs=[
            pl.BlockSpec((BN, D), lambda i: (i, 0)),
            pl.BlockSpec((BN, 16), lambda i: (i, 0)),
            pl.BlockSpec((BN, 16), lambda i: (i, 0)),
        ],
        out_shape=[
            jax.ShapeDtypeStruct((N, D), jnp.bfloat16),
            jax.ShapeDtypeStruct((N, 16), jnp.float32),
            jax.ShapeDtypeStruct((N, 16), jnp.float32),
        ],
    )(p, rmat, wflat, a1m, a2m)


def _tc_finish(p, rmat):
    return pl.pallas_call(
        _finish_body,
        grid=(N // BN,),
        in_specs=[
            pl.BlockSpec((2, BN, TD), lambda i: (0, i, 0)),
            pl.BlockSpec((H, D), lambda i: (0, 0)),
        ],
        out_specs=pl.BlockSpec((BN, D), lambda i: (i, 0)),
        out_shape=jax.ShapeDtypeStruct((N, D), jnp.float32),
    )(p, rmat)


def _sc_edges_body(src_hbm, dst_hbm, tdf_hbm, ts1_hbm, ts2_hbm, zero_hbm,
                   out_hbm,
                   s_idx0, d_idx0, rdf0, rs1_0, rs2_0, wrow0, c_idx0,
                   s_idx1, d_idx1, rdf1, rs1_1, rs2_1, wrow1, c_idx1,
                   acc,
                   semi0, semf0, sem10, sem20, semw0,
                   semi1, semf1, sem11, sem21, semw1):
    cid = lax.axis_index("c")
    sid = lax.axis_index("s")
    wid = cid * jnp.int32(NS) + sid
    row0 = sid * jnp.int32(ROWS_PER_TILE)

    bufs = ((s_idx0, d_idx0, rdf0, rs1_0, rs2_0, wrow0, c_idx0,
             semi0, semf0, sem10, sem20, semw0),
            (s_idx1, d_idx1, rdf1, rs1_1, rs2_1, wrow1, c_idx1,
             semi1, semf1, sem11, sem21, semw1))

    def issue_idx(k, b):
        s_idx, d_idx = bufs[b][0], bufs[b][1]
        semi = bufs[b][7]
        base = wid * jnp.int32(EPW) + k * jnp.int32(C)
        pltpu.async_copy(src_hbm.at[pl.ds(base, C)], s_idx, semi)
        pltpu.async_copy(dst_hbm.at[pl.ds(base, C)], d_idx, semi)

    def wait_idx(b):
        s_idx, d_idx = bufs[b][0], bufs[b][1]
        semi = bufs[b][7]
        pltpu.make_async_copy(src_hbm.at[pl.ds(0, C)], s_idx, semi).wait()
        pltpu.make_async_copy(dst_hbm.at[pl.ds(0, C)], d_idx, semi).wait()

    def issue_rows(b):
        s_idx, d_idx, rdf, rs1, rs2 = bufs[b][:5]
        semf, sem1, sem2 = bufs[b][8], bufs[b][9], bufs[b][10]
        pltpu.async_copy(tdf_hbm.at[d_idx], rdf, semf)
        pltpu.async_copy(ts1_hbm.at[s_idx], rs1, sem1)
        pltpu.async_copy(ts2_hbm.at[d_idx], rs2, sem2)

    def wait_rows(b):
        s_idx, d_idx, rdf, rs1, rs2 = bufs[b][:5]
        semf, sem1, sem2 = bufs[b][8], bufs[b][9], bufs[b][10]
        pltpu.make_async_copy(tdf_hbm.at[d_idx], rdf, semf).wait()
        pltpu.make_async_copy(ts1_hbm.at[s_idx], rs1, sem1).wait()
        pltpu.make_async_copy(ts2_hbm.at[d_idx], rs2, sem2).wait()

    def wait_scatter(b):
        wrow, c_idx, semw = bufs[b][5], bufs[b][6], bufs[b][11]
        pltpu.make_async_copy(wrow, acc.at[c_idx], semw).wait()

    def copy_scatter_idx(b):
        s_idx, c_idx = bufs[b][0], bufs[b][6]
        for r in range(C // 16):
            c_idx[pl.ds(r * 16, 16)] = s_idx[pl.ds(r * 16, 16)]

    def compute_scatter(b):
        _, _, rdf, rs1, rs2, wrow, c_idx = bufs[b][:7]
        semw = bufs[b][11]

        @plsc.parallel_loop(0, C)
        def edge(i):
            t = rs1[i, :] + rs2[i, :]
            lr = jnp.where(t >= 0, t, ALPHA * t)
            e = jnp.exp(-lr)
            wrow[i, pl.ds(128, 16)] = e
            for p in range(4):
                pair = rdf[i, pl.ds(p * 32, 32)]
                fa, fb = plsc.unpack(pair, format=plsc.PackFormat.INTERLEAVED)
                wrow[i, pl.ds(p * 32, 16)] = fa * e[2 * p]
                wrow[i, pl.ds(p * 32 + 16, 16)] = fb * e[2 * p + 1]

        pltpu.async_copy(wrow, acc.at[c_idx], semw, add=True)

    pltpu.sync_copy(zero_hbm, acc.at[pl.ds(row0, ROWS_PER_TILE)])
    plsc.subcore_barrier()

    issue_idx(jnp.int32(0), 0)
    issue_idx(jnp.int32(1), 1)
    wait_idx(0)
    issue_rows(0)

    def body2(j, carry):
        k = j * jnp.int32(2)

        wait_idx(1)
        issue_rows(1)
        wait_rows(0)

        @pl.when(j > 0)
        def _():
            wait_scatter(0)

        copy_scatter_idx(0)

        @pl.when(k + 2 < NCHUNK)
        def _():
            issue_idx(k + 2, 0)

        compute_scatter(0)

        @pl.when(k + 2 < NCHUNK)
        def _():
            wait_idx(0)
            issue_rows(0)

        wait_rows(1)

        @pl.when(j > 0)
        def _():
            wait_scatter(1)

        copy_scatter_idx(1)

        @pl.when(k + 3 < NCHUNK)
        def _():
            issue_idx(k + 3, 1)

        compute_scatter(1)
        return carry

    lax.fori_loop(jnp.int32(0), jnp.int32(NCHUNK // 2), body2, jnp.int32(0))
    wait_rows(0)
    wait_scatter(0)
    copy_scatter_idx(0)
    compute_scatter(0)
    wait_scatter(0)
    wait_scatter(1)
    plsc.subcore_barrier()
    pltpu.sync_copy(acc.at[pl.ds(row0, ROWS_PER_TILE)],
                    out_hbm.at[cid, pl.ds(row0, ROWS_PER_TILE)])


_sc_edges = functools.partial(
    pl.kernel,
    out_type=jax.ShapeDtypeStruct((NC, NPAD, TD), jnp.float32),
    mesh=plsc.VectorSubcoreMesh(core_axis_name="c", subcore_axis_name="s"),
    compiler_params=pltpu.CompilerParams(use_tc_tiling_on_sc=False,
                                         needs_layout_passes=False),
    scratch_types=[
        pltpu.VMEM((C,), jnp.int32),
        pltpu.VMEM((C,), jnp.int32),
        pltpu.VMEM((C, D), jnp.bfloat16),
        pltpu.VMEM((C, 16), jnp.float32),
        pltpu.VMEM((C, 16), jnp.float32),
        pltpu.VMEM((C, TD), jnp.float32),
        pltpu.VMEM((C,), jnp.int32),
        pltpu.VMEM((C,), jnp.int32),
        pltpu.VMEM((C,), jnp.int32),
        pltpu.VMEM((C, D), jnp.bfloat16),
        pltpu.VMEM((C, 16), jnp.float32),
        pltpu.VMEM((C, 16), jnp.float32),
        pltpu.VMEM((C, TD), jnp.float32),
        pltpu.VMEM((C,), jnp.int32),
        pltpu.VMEM_SHARED((NPAD, TD), jnp.float32),
        pltpu.SemaphoreType.DMA,
        pltpu.SemaphoreType.DMA,
        pltpu.SemaphoreType.DMA,
        pltpu.SemaphoreType.DMA,
        pltpu.SemaphoreType.DMA,
        pltpu.SemaphoreType.DMA,
        pltpu.SemaphoreType.DMA,
        pltpu.SemaphoreType.DMA,
        pltpu.SemaphoreType.DMA,
        pltpu.SemaphoreType.DMA,
    ],
)(_sc_edges_body)


def _layer_weights(Wl, al):
    wflat = jnp.transpose(Wl, (1, 0, 2)).reshape(D, H * DH)[:, _PERM]
    a1 = al[:, :DH]
    a2 = al[:, DH:]
    eye = jnp.eye(H, dtype=jnp.float32)
    a1m = (a1[:, :, None] * eye[:, None, :]).reshape(H * DH, H)[_PERM, :]
    a2m = (a2[:, :, None] * eye[:, None, :]).reshape(H * DH, H)[_PERM, :]
    return wflat, a1m, a2m


def kernel(x, adj, W, a):
    with _config.enable_x64(False):
        x = x.astype(jnp.float32)
        src = adj[0].astype(jnp.int32)
        dst = adj[1].astype(jnp.int32)
        W = W.astype(jnp.float32)
        a = a.astype(jnp.float32)
        zero = jnp.zeros((ROWS_PER_TILE, TD), jnp.float32)
        rmat = jnp.repeat(jnp.eye(H, dtype=jnp.float32), DH, axis=1)

        w0, a1m0, a2m0 = _layer_weights(W[0], a[0])
        w1, a1m1, a2m1 = _layer_weights(W[1], a[1])

        tdf, ts1, ts2 = _tc_proj(x, w0, a1m0, a2m0)
        p = _sc_edges(src, dst, tdf, ts1, ts2, zero)
        tdf, ts1, ts2 = _tc_finish_proj(p, rmat, w1, a1m1, a2m1)
        p = _sc_edges(src, dst, tdf, ts1, ts2, zero)
        return _tc_finish(p, rmat)
